# Initial kernel scaffold; baseline (speedup 1.0000x reference)
import jax
import jax.numpy as jnp
from jax.experimental import pallas as pl


def kernel(position_ids, y_table):
    raise NotImplementedError("write your pallas kernel here")


# speedup vs baseline: 5.5750x; 5.5750x over previous
import functools

import jax
import jax.numpy as jnp
from jax import lax
from jax.experimental import pallas as pl
from jax.experimental.pallas import tpu as pltpu
from jax.experimental.pallas import tpu_sc as plsc

X_SHAPE = 1000
Y_SHAPE = 100000

NW = 32
CH = 512
ISUB = 128
NSUB = CH // ISUB


@functools.lru_cache(maxsize=None)
def _build(tot, vocab, dim):
    per_w = tot // NW
    nch = per_w // CH
    assert per_w % CH == 0 and dim % 16 == 0

    mesh = plsc.VectorSubcoreMesh(core_axis_name="c", subcore_axis_name="s")

    @functools.partial(
        pl.kernel,
        mesh=mesh,
        compiler_params=pltpu.CompilerParams(use_tc_tiling_on_sc=False),
        out_type=jax.ShapeDtypeStruct((tot, dim), jnp.float32),
        scratch_types=[
            pltpu.VMEM((CH,), jnp.int32),
            pltpu.VMEM((NSUB, ISUB), jnp.int32),
            pltpu.VMEM((NSUB, ISUB), jnp.int32),
            pltpu.VMEM((CH, dim), jnp.float32),
            pltpu.VMEM((CH, dim), jnp.float32),
            pltpu.SemaphoreType.DMA,
        ],
    )
    def k(pid_hbm, table_hbm, out_hbm, pid_v, row_v, col_v, buf1, buf2, gsem):
        wid = lax.axis_index("s") * 2 + lax.axis_index("c")
        base = wid * per_w
        xs = jnp.full((16,), X_SHAPE, jnp.int32)
        ys = jnp.full((16,), Y_SHAPE, jnp.int32)

        def chunk(ci, carry):
            off = base + ci * CH
            pltpu.sync_copy(pid_hbm.at[pl.ds(off, CH)], pid_v)

            def conv(i, c):
                p = pid_v[pl.ds(i * 16, 16)]
                row_v[i // 8, pl.ds((i % 8) * 16, 16)] = lax.div(p, xs)
                col_v[i // 8, pl.ds((i % 8) * 16, 16)] = lax.rem(p, ys)
                return c

            lax.fori_loop(0, CH // 16, conv, 0)

            copies = []
            for j in range(NSUB):
                copies.append(pltpu.async_copy(
                    table_hbm.at[row_v.at[j]], buf1.at[pl.ds(j * ISUB, ISUB)], gsem))
                copies.append(pltpu.async_copy(
                    table_hbm.at[col_v.at[j]], buf2.at[pl.ds(j * ISUB, ISUB)], gsem))
            for c in copies:
                c.wait()

            def addl(i, c):
                for d in range(dim // 16):
                    sl = pl.ds(d * 16, 16)
                    buf1[i, sl] = buf1[i, sl] + buf2[i, sl]
                return c

            lax.fori_loop(0, CH, addl, 0)
            pltpu.sync_copy(buf1, out_hbm.at[pl.ds(off, CH)])
            return carry

        lax.fori_loop(0, nch, chunk, 0)

    return k


def kernel(position_ids, y_table):
    b, h = position_ids.shape
    vocab, dim = y_table.shape
    pid = position_ids.reshape(-1).astype(jnp.int32)
    out = _build(b * h, vocab, dim)(pid, y_table)
    return out.reshape(b, h, dim)


# speedup vs baseline: 6.0558x; 1.0862x over previous
import functools

import jax
import jax.numpy as jnp
from jax import lax
from jax.experimental import pallas as pl
from jax.experimental.pallas import tpu as pltpu
from jax.experimental.pallas import tpu_sc as plsc

X_SHAPE = 1000
Y_SHAPE = 100000

NW = 32
CH = 512
ISUB = 128
NSUB = CH // ISUB


@functools.lru_cache(maxsize=None)
def _build(tot, vocab, dim):
    per_w = tot // NW
    nch = per_w // CH
    assert per_w % CH == 0 and dim % 16 == 0

    mesh = plsc.VectorSubcoreMesh(core_axis_name="c", subcore_axis_name="s")

    @functools.partial(
        pl.kernel,
        mesh=mesh,
        compiler_params=pltpu.CompilerParams(use_tc_tiling_on_sc=False),
        out_type=jax.ShapeDtypeStruct((tot, dim), jnp.float32),
        scratch_types=[
            pltpu.VMEM((CH,), jnp.int32),
            pltpu.VMEM((NSUB, ISUB), jnp.int32),
            pltpu.VMEM((NSUB, ISUB), jnp.int32),
            pltpu.VMEM((CH, dim), jnp.float32),
            pltpu.VMEM((CH, dim), jnp.float32),
            pltpu.SemaphoreType.DMA,
        ],
    )
    def k(pid_hbm, table_hbm, out_hbm, pid_v, row_v, col_v, buf1, buf2, gsem):
        wid = lax.axis_index("s") * 2 + lax.axis_index("c")
        base = wid * per_w
        xs = jnp.full((16,), X_SHAPE, jnp.int32)
        ys = jnp.full((16,), Y_SHAPE, jnp.int32)

        def chunk(ci, carry):
            off = base + ci * CH
            pltpu.sync_copy(pid_hbm.at[pl.ds(off, CH)], pid_v)

            def conv(i, c):
                p = pid_v[pl.ds(i * 16, 16)]
                row_v[i // 8, pl.ds((i % 8) * 16, 16)] = lax.div(p, xs)
                col_v[i // 8, pl.ds((i % 8) * 16, 16)] = lax.rem(p, ys)
                return c

            lax.fori_loop(0, CH // 16, conv, 0)

            copies = []
            for j in range(NSUB):
                copies.append(pltpu.async_copy(
                    table_hbm.at[row_v.at[j]], buf1.at[pl.ds(j * ISUB, ISUB)], gsem))
            for c in copies:
                c.wait()
            copies = []
            for j in range(NSUB):
                copies.append(pltpu.async_copy(
                    table_hbm.at[col_v.at[j]], buf1.at[pl.ds(j * ISUB, ISUB)],
                    gsem, add=True))
            for c in copies:
                c.wait()
            pltpu.sync_copy(buf1, out_hbm.at[pl.ds(off, CH)])
            return carry

        lax.fori_loop(0, nch, chunk, 0)

    return k


def kernel(position_ids, y_table):
    b, h = position_ids.shape
    vocab, dim = y_table.shape
    pid = position_ids.reshape(-1).astype(jnp.int32)
    out = _build(b * h, vocab, dim)(pid, y_table)
    return out.reshape(b, h, dim)


# speedup vs baseline: 6.5732x; 1.0854x over previous
import functools

import jax
import jax.numpy as jnp
from jax import lax
from jax.experimental import pallas as pl
from jax.experimental.pallas import tpu as pltpu
from jax.experimental.pallas import tpu_sc as plsc

X_SHAPE = 1000
Y_SHAPE = 100000

NW = 32
CH = 512
ISUB = 128
NSUB = CH // ISUB


@functools.lru_cache(maxsize=None)
def _build(tot, vocab, dim):
    per_w = tot // NW
    nch = per_w // CH
    assert per_w % CH == 0 and nch % 2 == 0 and dim % 16 == 0

    mesh = plsc.VectorSubcoreMesh(core_axis_name="c", subcore_axis_name="s")

    @functools.partial(
        pl.kernel,
        mesh=mesh,
        compiler_params=pltpu.CompilerParams(use_tc_tiling_on_sc=False),
        out_type=jax.ShapeDtypeStruct((tot, dim), jnp.float32),
        scratch_types=[
            pltpu.VMEM((CH,), jnp.int32),
            pltpu.VMEM((NSUB, ISUB), jnp.int32),
            pltpu.VMEM((NSUB, ISUB), jnp.int32),
            pltpu.VMEM((NSUB, ISUB), jnp.int32),
            pltpu.VMEM((NSUB, ISUB), jnp.int32),
            pltpu.VMEM((CH, dim), jnp.float32),
            pltpu.VMEM((CH, dim), jnp.float32),
            pltpu.SemaphoreType.DMA,
            pltpu.SemaphoreType.DMA,
            pltpu.SemaphoreType.DMA,
        ],
    )
    def k(pid_hbm, table_hbm, out_hbm, pid_v,
          row_v0, row_v1, col_v0, col_v1, buf0, buf1, gsB, gsC, osem):
        wid = lax.axis_index("s") * 2 + lax.axis_index("c")
        base = wid * per_w
        row_v = (row_v0, row_v1)
        col_v = (col_v0, col_v1)
        buf = (buf0, buf1)
        xs = jnp.full((16,), X_SHAPE, jnp.int32)
        ys = jnp.full((16,), Y_SHAPE, jnp.int32)

        def conv(ci, s):
            pltpu.sync_copy(pid_hbm.at[pl.ds(base + ci * CH, CH)], pid_v)

            def body(i, c):
                p = pid_v[pl.ds(i * 16, 16)]
                row_v[s][i // 8, pl.ds((i % 8) * 16, 16)] = lax.div(p, xs)
                col_v[s][i // 8, pl.ds((i % 8) * 16, 16)] = lax.rem(p, ys)
                return c

            lax.fori_loop(0, CH // 16, body, 0)

        def fire_b(s):
            for j in range(NSUB):
                pltpu.async_copy(
                    table_hbm.at[row_v[s].at[j]],
                    buf[s].at[pl.ds(j * ISUB, ISUB)], gsB)

        def wait_b(s):
            for j in range(NSUB):
                pltpu.make_async_copy(
                    table_hbm.at[row_v[s].at[j]],
                    buf[s].at[pl.ds(j * ISUB, ISUB)], gsB).wait()

        def fire_c(s):
            for j in range(NSUB):
                pltpu.async_copy(
                    table_hbm.at[col_v[s].at[j]],
                    buf[s].at[pl.ds(j * ISUB, ISUB)], gsC, add=True)

        def wait_c(s):
            for j in range(NSUB):
                pltpu.make_async_copy(
                    table_hbm.at[col_v[s].at[j]],
                    buf[s].at[pl.ds(j * ISUB, ISUB)], gsC).wait()

        def fire_d(ci, s):
            pltpu.async_copy(buf[s], out_hbm.at[pl.ds(base + ci * CH, CH)], osem)

        def wait_d(s):
            pltpu.make_async_copy(
                buf[s], out_hbm.at[pl.ds(base, CH)], osem).wait()

        conv(0, 0)
        fire_b(0)
        conv(1, 1)
        wait_b(0)
        fire_c(0)
        fire_b(1)
        wait_c(0)
        fire_d(0, 0)

        def pair(p, carry):
            c0 = 2 * p
            c1 = c0 + 1
            conv(c0, 0)
            wait_b(1)
            fire_c(1)
            wait_d(0)
            fire_b(0)
            wait_c(1)
            fire_d(c0 - 1, 1)
            conv(c1, 1)
            wait_b(0)
            fire_c(0)
            wait_d(1)
            fire_b(1)
            wait_c(0)
            fire_d(c0, 0)
            return carry

        lax.fori_loop(1, nch // 2, pair, 0)

        wait_b(1)
        fire_c(1)
        wait_d(0)
        wait_c(1)
        fire_d(nch - 1, 1)
        wait_d(1)

    return k


def kernel(position_ids, y_table):
    b, h = position_ids.shape
    vocab, dim = y_table.shape
    pid = position_ids.reshape(-1).astype(jnp.int32)
    out = _build(b * h, vocab, dim)(pid, y_table)
    return out.reshape(b, h, dim)


# speedup vs baseline: 6.7259x; 1.0232x over previous
import functools

import jax
import jax.numpy as jnp
from jax import lax
from jax.experimental import pallas as pl
from jax.experimental.pallas import tpu as pltpu
from jax.experimental.pallas import tpu_sc as plsc

X_SHAPE = 1000
Y_SHAPE = 100000

NW = 32
ROWS_PER_CHUNK = 4
ISUB = 80


@functools.lru_cache(maxsize=None)
def _build(nb, hist, vocab, dim):
    ch = ROWS_PER_CHUNK * hist
    nsub = ch // ISUB
    per_w = (nb * hist) // NW
    nch = per_w // ch
    rows_w = nb // NW
    assert per_w % ch == 0 and nch % 2 == 0 and dim % 16 == 0
    assert ch % ISUB == 0 and ch % 16 == 0 and hist % 8 == 0

    mesh = plsc.VectorSubcoreMesh(core_axis_name="c", subcore_axis_name="s")

    @functools.partial(
        pl.kernel,
        mesh=mesh,
        compiler_params=pltpu.CompilerParams(use_tc_tiling_on_sc=False),
        out_type=jax.ShapeDtypeStruct((nb, hist, dim), jnp.float32),
        scratch_types=[
            pltpu.VMEM((ch,), jnp.int32),
            pltpu.VMEM((ch,), jnp.int32),
            pltpu.VMEM((ch,), jnp.int32),
            pltpu.VMEM((ch,), jnp.int32),
            pltpu.VMEM((ch,), jnp.int32),
            pltpu.VMEM((ch, dim), jnp.float32),
            pltpu.VMEM((ch, dim), jnp.float32),
            pltpu.SemaphoreType.DMA,
            pltpu.SemaphoreType.DMA,
            pltpu.SemaphoreType.DMA,
        ],
    )
    def k(pid_hbm, table_hbm, out_hbm, pid_v,
          row_v0, row_v1, col_v0, col_v1, buf0, buf1, gsB, gsC, osem):
        wid = lax.axis_index("s") * 2 + lax.axis_index("c")
        base = wid * per_w
        brow = wid * rows_w
        row_v = (row_v0, row_v1)
        col_v = (col_v0, col_v1)
        buf = (buf0, buf1)
        xs = jnp.full((16,), X_SHAPE, jnp.int32)
        ys = jnp.full((16,), Y_SHAPE, jnp.int32)

        def conv(ci, s):
            pltpu.sync_copy(pid_hbm.at[pl.ds(base + ci * ch, ch)], pid_v)

            def body(i, c):
                sl = pl.ds(i * 16, 16)
                p = pid_v[sl]
                row_v[s][sl] = lax.div(p, xs)
                col_v[s][sl] = lax.rem(p, ys)
                return c

            lax.fori_loop(0, ch // 16, body, 0)

        def fire_b(s):
            for j in range(nsub):
                pltpu.async_copy(
                    table_hbm.at[row_v[s].at[pl.ds(j * ISUB, ISUB)]],
                    buf[s].at[pl.ds(j * ISUB, ISUB)], gsB)

        def wait_b(s):
            for j in range(nsub):
                pltpu.make_async_copy(
                    table_hbm.at[row_v[s].at[pl.ds(j * ISUB, ISUB)]],
                    buf[s].at[pl.ds(j * ISUB, ISUB)], gsB).wait()

        def fire_c(s):
            for j in range(nsub):
                pltpu.async_copy(
                    table_hbm.at[col_v[s].at[pl.ds(j * ISUB, ISUB)]],
                    buf[s].at[pl.ds(j * ISUB, ISUB)], gsC, add=True)

        def wait_c(s):
            for j in range(nsub):
                pltpu.make_async_copy(
                    table_hbm.at[col_v[s].at[pl.ds(j * ISUB, ISUB)]],
                    buf[s].at[pl.ds(j * ISUB, ISUB)], gsC).wait()

        def fire_d(ci, s):
            for r in range(ROWS_PER_CHUNK):
                pltpu.async_copy(
                    buf[s].at[pl.ds(r * hist, hist)],
                    out_hbm.at[brow + ci * ROWS_PER_CHUNK + r], osem)

        def wait_d(s):
            for r in range(ROWS_PER_CHUNK):
                pltpu.make_async_copy(
                    buf[s].at[pl.ds(r * hist, hist)],
                    out_hbm.at[brow + r], osem).wait()

        conv(0, 0)
        fire_b(0)
        conv(1, 1)
        wait_b(0)
        fire_c(0)
        fire_b(1)
        wait_c(0)
        fire_d(0, 0)

        def pair(p, carry):
            c0 = 2 * p
            c1 = c0 + 1
            conv(c0, 0)
            wait_b(1)
            fire_c(1)
            wait_d(0)
            fire_b(0)
            wait_c(1)
            fire_d(c0 - 1, 1)
            conv(c1, 1)
            wait_b(0)
            fire_c(0)
            wait_d(1)
            fire_b(1)
            wait_c(0)
            fire_d(c0, 0)
            return carry

        lax.fori_loop(1, nch // 2, pair, 0)

        wait_b(1)
        fire_c(1)
        wait_d(0)
        wait_c(1)
        fire_d(nch - 1, 1)
        wait_d(1)

    return k


def kernel(position_ids, y_table):
    nb, hist = position_ids.shape
    vocab, dim = y_table.shape
    pid = position_ids.reshape(-1)
    return _build(nb, hist, vocab, dim)(pid, y_table)
